# Initial kernel scaffold; baseline (speedup 1.0000x reference)
import jax
import jax.numpy as jnp
from jax.experimental import pallas as pl


def kernel(x, edge_index, W1, a_src1, a_dst1, b1, W2, a_src2, a_dst2, b2):
    raise NotImplementedError("write your pallas kernel here")


# speedup vs baseline: 38.9224x; 38.9224x over previous
import dataclasses
import functools

import jax
import jax.numpy as jnp
from jax import lax
from jax.experimental import pallas as pl
from jax.experimental.pallas import tpu as pltpu
from jax.experimental.pallas import tpu_sc as plsc

N = 10000
E = 320000
NC = 2
NS = 16
NW = NC * NS
EPW = E // NW
EB = 80
NBLK = EPW // EB
TW = 80
ZR = 40
NCH = N // ZR
CIT = -(-NCH // NS)

_HI = jax.lax.Precision.HIGHEST
_f32 = jnp.float32


def _leaky_exp(s):
    return jnp.exp(jnp.maximum(s, s * jnp.float32(0.2)))


_R = 2000
_G = N // _R


def _tc_stage_a(x, W1, A_s, A_d):
    def body(x_ref, w_ref, as_ref, ad_ref, t_ref, adt_ref):
        h = lax.dot(x_ref[...], w_ref[...], precision=_HI,
                    preferred_element_type=_f32)
        als = lax.dot(h, as_ref[...], precision=_HI,
                      preferred_element_type=_f32)
        ald = lax.dot(h, ad_ref[...], precision=_HI,
                      preferred_element_type=_f32)
        z8 = jnp.zeros((_R, 8), _f32)
        t_ref[...] = jnp.concatenate([h, als, z8], axis=1)
        adt_ref[...] = jnp.concatenate([ald, z8], axis=1)

    return pl.pallas_call(
        body,
        grid=(_G,),
        in_specs=[
            pl.BlockSpec((_R, 128), lambda i: (i, 0)),
            pl.BlockSpec((128, 64), lambda i: (0, 0)),
            pl.BlockSpec((64, 8), lambda i: (0, 0)),
            pl.BlockSpec((64, 8), lambda i: (0, 0)),
        ],
        out_specs=[
            pl.BlockSpec((_R, TW), lambda i: (i, 0)),
            pl.BlockSpec((_R, 16), lambda i: (i, 0)),
        ],
        out_shape=[
            jax.ShapeDtypeStruct((N, TW), _f32),
            jax.ShapeDtypeStruct((N, 16), _f32),
        ],
    )(x, W1, A_s, A_d)


def _tc_stage_b(accn, accd, t1, ad1, b1, W2, a2s, a2d):
    def body(an_ref, adn_ref, t_ref, adt_ref, b_ref, w_ref, a2s_ref, a2d_ref,
             t2_ref, adt2_ref):
        nb = an_ref[0] + an_ref[1]
        db = adn_ref[0] + adn_ref[1]
        h1 = t_ref[:, :64]
        as1 = t_ref[:, 64:72]
        ad1v = adt_ref[:, :8]
        wself = _leaky_exp(as1 + ad1v)
        parts = []
        for hh in range(8):
            ws = wself[:, hh:hh + 1]
            numh = nb[:, 8 * hh:8 * hh + 8] + h1[:, 8 * hh:8 * hh + 8] * ws
            denh = db[:, hh:hh + 1] + ws
            parts.append(numh / denh)
        out1 = jnp.concatenate(parts, axis=1) + b_ref[...]
        x2 = jnp.where(out1 > 0, out1, jnp.exp(out1) - jnp.float32(1.0))
        h2 = lax.dot(x2, w_ref[...], precision=_HI,
                     preferred_element_type=_f32)
        as2 = jnp.sum(h2 * a2s_ref[...], axis=1, keepdims=True)
        ad2 = jnp.sum(h2 * a2d_ref[...], axis=1, keepdims=True)
        z15 = jnp.zeros((_R, 15), _f32)
        t2_ref[...] = jnp.concatenate([h2, as2, z15], axis=1)
        adt2_ref[...] = jnp.concatenate([ad2, z15], axis=1)

    return pl.pallas_call(
        body,
        grid=(_G,),
        in_specs=[
            pl.BlockSpec((NC, _R, 64), lambda i: (0, i, 0)),
            pl.BlockSpec((NC, _R, 16), lambda i: (0, i, 0)),
            pl.BlockSpec((_R, TW), lambda i: (i, 0)),
            pl.BlockSpec((_R, 16), lambda i: (i, 0)),
            pl.BlockSpec((1, 64), lambda i: (0, 0)),
            pl.BlockSpec((64, 64), lambda i: (0, 0)),
            pl.BlockSpec((1, 64), lambda i: (0, 0)),
            pl.BlockSpec((1, 64), lambda i: (0, 0)),
        ],
        out_specs=[
            pl.BlockSpec((_R, TW), lambda i: (i, 0)),
            pl.BlockSpec((_R, 16), lambda i: (i, 0)),
        ],
        out_shape=[
            jax.ShapeDtypeStruct((N, TW), _f32),
            jax.ShapeDtypeStruct((N, 16), _f32),
        ],
    )(accn, accd, t1, ad1, b1, W2, a2s, a2d)


def _tc_stage_c(accn, accd, t2, ad2, b2):
    def body(an_ref, adn_ref, t_ref, adt_ref, b_ref, o_ref):
        nb = an_ref[0] + an_ref[1]
        db = adn_ref[0][:, :1] + adn_ref[1][:, :1]
        h2 = t_ref[:, :64]
        as2 = t_ref[:, 64:65]
        ad2v = adt_ref[:, :1]
        wself = _leaky_exp(as2 + ad2v)
        o_ref[...] = (nb + h2 * wself) / (db + wself) + b_ref[...]

    return pl.pallas_call(
        body,
        grid=(_G,),
        in_specs=[
            pl.BlockSpec((NC, _R, 64), lambda i: (0, i, 0)),
            pl.BlockSpec((NC, _R, 16), lambda i: (0, i, 0)),
            pl.BlockSpec((_R, TW), lambda i: (i, 0)),
            pl.BlockSpec((_R, 16), lambda i: (i, 0)),
            pl.BlockSpec((1, 64), lambda i: (0, 0)),
        ],
        out_specs=pl.BlockSpec((_R, 64), lambda i: (i, 0)),
        out_shape=jax.ShapeDtypeStruct((N, 64), _f32),
    )(accn, accd, t2, ad2, b2)


def _sc_compiler_params():
    cp = pltpu.CompilerParams()
    fields = pltpu.CompilerParams.__dataclass_fields__
    if "needs_layout_passes" in fields:
        cp = dataclasses.replace(cp, needs_layout_passes=False)
    if "use_tc_tiling_on_sc" in fields:
        cp = dataclasses.replace(cp, use_tc_tiling_on_sc=False)
    return cp


@functools.partial(jax.jit, static_argnames=("shift",))
def _sc_edge_pass(t_arr, ad_arr, src, dst, *, shift):
    mesh = plsc.VectorSubcoreMesh(core_axis_name="c", subcore_axis_name="s",
                                  num_cores=NC, num_subcores=NS)

    def body(t_hbm, ad_hbm, src_hbm, dst_hbm, accn_hbm, accd_hbm,
             t_sh, ad_sh, accn_sh, accd_sh,
             srcv, dstv, trows, adrows, wbuf, outb, zbuf, zbufd):
        cid = lax.axis_index("c")
        sid = lax.axis_index("s")
        wid = sid * NC + cid

        z16 = jnp.zeros((16,), _f32)

        @pl.loop(0, ZR)
        def _(r):
            for c in range(4):
                zbuf[r, pl.ds(16 * c, 16)] = z16
            zbufd[r, pl.ds(0, 16)] = z16

        @pl.loop(0, CIT)
        def _(i):
            c = i * NS + sid

            @pl.when(c < NCH)
            def _():
                r0 = c * ZR
                pltpu.sync_copy(zbuf, accn_sh.at[pl.ds(r0, ZR)])
                pltpu.sync_copy(zbufd, accd_sh.at[pl.ds(r0, ZR)])
                pltpu.sync_copy(t_hbm.at[pl.ds(r0, ZR)],
                                t_sh.at[pl.ds(r0, ZR)])
                pltpu.sync_copy(ad_hbm.at[pl.ds(r0, ZR)],
                                ad_sh.at[pl.ds(r0, ZR)])

        plsc.subcore_barrier()

        iot = lax.iota(jnp.int32, 16)
        pks = [lax.shift_right_logical(iot + (16 * k), shift)
               for k in range(4)]

        ebase = wid * EPW

        @pl.loop(0, NBLK)
        def _(k):
            b0 = ebase + k * EB
            pltpu.sync_copy(src_hbm.at[pl.ds(b0, EB)], srcv)
            pltpu.sync_copy(dst_hbm.at[pl.ds(b0, EB)], dstv)
            pltpu.sync_copy(t_sh.at[srcv], trows)
            pltpu.sync_copy(ad_sh.at[dstv], adrows)

            @pl.loop(0, EB)
            def _(b):
                s = trows[b, pl.ds(64, 16)] + adrows[b, pl.ds(0, 16)]
                w = _leaky_exp(s)
                wbuf[b, pl.ds(0, 16)] = w
                rowi = jnp.full((16,), b, jnp.int32)
                for kk in range(4):
                    wb = plsc.load_gather(wbuf, [rowi, pks[kk]])
                    outb[b, pl.ds(16 * kk, 16)] = (
                        trows[b, pl.ds(16 * kk, 16)] * wb)

            pltpu.sync_copy(outb, accn_sh.at[dstv], add=True)
            pltpu.sync_copy(wbuf, accd_sh.at[dstv], add=True)

        plsc.subcore_barrier()

        @pl.loop(0, CIT)
        def _(i):
            c = i * NS + sid

            @pl.when(c < NCH)
            def _():
                r0 = c * ZR
                pltpu.sync_copy(accn_sh.at[pl.ds(r0, ZR)],
                                accn_hbm.at[cid, pl.ds(r0, ZR)])
                pltpu.sync_copy(accd_sh.at[pl.ds(r0, ZR)],
                                accd_hbm.at[cid, pl.ds(r0, ZR)])

    kern = pl.kernel(
        body,
        out_type=(
            jax.ShapeDtypeStruct((NC, N, 64), _f32),
            jax.ShapeDtypeStruct((NC, N, 16), _f32),
        ),
        mesh=mesh,
        scratch_types=[
            pltpu.VMEM_SHARED((N, TW), _f32),
            pltpu.VMEM_SHARED((N, 16), _f32),
            pltpu.VMEM_SHARED((N, 64), _f32),
            pltpu.VMEM_SHARED((N, 16), _f32),
            pltpu.VMEM((EB,), jnp.int32),
            pltpu.VMEM((EB,), jnp.int32),
            pltpu.VMEM((EB, TW), _f32),
            pltpu.VMEM((EB, 16), _f32),
            pltpu.VMEM((EB, 16), _f32),
            pltpu.VMEM((EB, 64), _f32),
            pltpu.VMEM((ZR, 64), _f32),
            pltpu.VMEM((ZR, 16), _f32),
        ],
        compiler_params=_sc_compiler_params(),
    )
    return kern(t_arr, ad_arr, src, dst)


def kernel(x, edge_index, W1, a_src1, a_dst1, b1, W2, a_src2, a_dst2, b2):
    src = edge_index[0]
    dst = edge_index[1]

    eye8 = jnp.eye(8, dtype=_f32)
    A_s1 = (a_src1.reshape(8, 8)[:, :, None] * eye8[:, None, :]).reshape(64, 8)
    A_d1 = (a_dst1.reshape(8, 8)[:, :, None] * eye8[:, None, :]).reshape(64, 8)

    t1, ad1 = _tc_stage_a(x, W1, A_s1, A_d1)
    accn1, accd1 = _sc_edge_pass(t1, ad1, src, dst, shift=3)
    t2, ad2 = _tc_stage_b(accn1, accd1, t1, ad1, b1.reshape(1, 64), W2,
                          a_src2.reshape(1, 64), a_dst2.reshape(1, 64))
    accn2, accd2 = _sc_edge_pass(t2, ad2, src, dst, shift=6)
    return _tc_stage_c(accn2, accd2, t2, ad2, b2.reshape(1, 64))


# speedup vs baseline: 42.8350x; 1.1005x over previous
import dataclasses
import functools

import jax
import jax.numpy as jnp
from jax import lax
from jax.experimental import pallas as pl
from jax.experimental.pallas import tpu as pltpu
from jax.experimental.pallas import tpu_sc as plsc

N = 10000
E = 320000
NC = 2
NS = 16
NW = NC * NS
EPW = E // NW
EB = 100
NBLK = EPW // EB
TW = 80
ZR = 16
NCH = N // ZR
CIT = -(-NCH // NS)
RPS = N // NS

_HI = jax.lax.Precision.HIGHEST
_f32 = jnp.float32


def _leaky_exp(s):
    return jnp.exp(jnp.maximum(s, s * jnp.float32(0.2)))


_R = 2000
_G = N // _R


def _tc_stage_a(x, W1, A_s, A_d):
    def body(x_ref, w_ref, as_ref, ad_ref, t_ref, adt_ref):
        h = lax.dot(x_ref[...], w_ref[...], precision=_HI,
                    preferred_element_type=_f32)
        als = lax.dot(h, as_ref[...], precision=_HI,
                      preferred_element_type=_f32)
        ald = lax.dot(h, ad_ref[...], precision=_HI,
                      preferred_element_type=_f32)
        z8 = jnp.zeros((_R, 8), _f32)
        t_ref[...] = jnp.concatenate([h, als, z8], axis=1)
        adt_ref[...] = jnp.concatenate([ald, z8], axis=1)

    return pl.pallas_call(
        body,
        grid=(_G,),
        in_specs=[
            pl.BlockSpec((_R, 128), lambda i: (i, 0)),
            pl.BlockSpec((128, 64), lambda i: (0, 0)),
            pl.BlockSpec((64, 8), lambda i: (0, 0)),
            pl.BlockSpec((64, 8), lambda i: (0, 0)),
        ],
        out_specs=[
            pl.BlockSpec((_R, TW), lambda i: (i, 0)),
            pl.BlockSpec((_R, 16), lambda i: (i, 0)),
        ],
        out_shape=[
            jax.ShapeDtypeStruct((N, TW), _f32),
            jax.ShapeDtypeStruct((N, 16), _f32),
        ],
    )(x, W1, A_s, A_d)


def _tc_stage_b(acc, t1, ad1, b1, W2, a2s, a2d):
    def body(a_ref, t_ref, adt_ref, b_ref, w_ref, a2s_ref, a2d_ref,
             t2_ref, adt2_ref):
        nb = a_ref[0, :, :64] + a_ref[1, :, :64]
        db = a_ref[0, :, 64:72] + a_ref[1, :, 64:72]
        h1 = t_ref[:, :64]
        as1 = t_ref[:, 64:72]
        ad1v = adt_ref[:, :8]
        wself = _leaky_exp(as1 + ad1v)
        parts = []
        for hh in range(8):
            ws = wself[:, hh:hh + 1]
            numh = nb[:, 8 * hh:8 * hh + 8] + h1[:, 8 * hh:8 * hh + 8] * ws
            denh = db[:, hh:hh + 1] + ws
            parts.append(numh / denh)
        out1 = jnp.concatenate(parts, axis=1) + b_ref[...]
        x2 = jnp.where(out1 > 0, out1, jnp.exp(out1) - jnp.float32(1.0))
        h2 = lax.dot(x2, w_ref[...], precision=_HI,
                     preferred_element_type=_f32)
        as2 = jnp.sum(h2 * a2s_ref[...], axis=1, keepdims=True)
        ad2 = jnp.sum(h2 * a2d_ref[...], axis=1, keepdims=True)
        z15 = jnp.zeros((_R, 15), _f32)
        t2_ref[...] = jnp.concatenate([h2, as2, z15], axis=1)
        adt2_ref[...] = jnp.concatenate([ad2, z15], axis=1)

    return pl.pallas_call(
        body,
        grid=(_G,),
        in_specs=[
            pl.BlockSpec((NC, _R, TW), lambda i: (0, i, 0)),
            pl.BlockSpec((_R, TW), lambda i: (i, 0)),
            pl.BlockSpec((_R, 16), lambda i: (i, 0)),
            pl.BlockSpec((1, 64), lambda i: (0, 0)),
            pl.BlockSpec((64, 64), lambda i: (0, 0)),
            pl.BlockSpec((1, 64), lambda i: (0, 0)),
            pl.BlockSpec((1, 64), lambda i: (0, 0)),
        ],
        out_specs=[
            pl.BlockSpec((_R, TW), lambda i: (i, 0)),
            pl.BlockSpec((_R, 16), lambda i: (i, 0)),
        ],
        out_shape=[
            jax.ShapeDtypeStruct((N, TW), _f32),
            jax.ShapeDtypeStruct((N, 16), _f32),
        ],
    )(acc, t1, ad1, b1, W2, a2s, a2d)


def _tc_stage_c(acc, t2, ad2, b2):
    def body(a_ref, t_ref, adt_ref, b_ref, o_ref):
        nb = a_ref[0, :, :64] + a_ref[1, :, :64]
        db = a_ref[0, :, 64:65] + a_ref[1, :, 64:65]
        h2 = t_ref[:, :64]
        as2 = t_ref[:, 64:65]
        ad2v = adt_ref[:, :1]
        wself = _leaky_exp(as2 + ad2v)
        o_ref[...] = (nb + h2 * wself) / (db + wself) + b_ref[...]

    return pl.pallas_call(
        body,
        grid=(_G,),
        in_specs=[
            pl.BlockSpec((NC, _R, TW), lambda i: (0, i, 0)),
            pl.BlockSpec((_R, TW), lambda i: (i, 0)),
            pl.BlockSpec((_R, 16), lambda i: (i, 0)),
            pl.BlockSpec((1, 64), lambda i: (0, 0)),
        ],
        out_specs=pl.BlockSpec((_R, 64), lambda i: (i, 0)),
        out_shape=jax.ShapeDtypeStruct((N, 64), _f32),
    )(acc, t2, ad2, b2)


def _sc_compiler_params():
    cp = pltpu.CompilerParams()
    fields = pltpu.CompilerParams.__dataclass_fields__
    if "needs_layout_passes" in fields:
        cp = dataclasses.replace(cp, needs_layout_passes=False)
    if "use_tc_tiling_on_sc" in fields:
        cp = dataclasses.replace(cp, use_tc_tiling_on_sc=False)
    return cp


@functools.partial(jax.jit, static_argnames=("shift",))
def _sc_edge_pass(t_arr, ad_arr, src, dst, *, shift):
    mesh = plsc.VectorSubcoreMesh(core_axis_name="c", subcore_axis_name="s",
                                  num_cores=NC, num_subcores=NS)

    def body(t_hbm, ad_hbm, src_hbm, dst_hbm, acc_hbm,
             t_sh, acc_sh, sidx, didx, outw, adrows):
        cid = lax.axis_index("c")
        sid = lax.axis_index("s")
        wid = sid * NC + cid

        z16 = jnp.zeros((16,), _f32)

        @pl.loop(0, EB)
        def _(r):
            for c in range(5):
                outw[r, pl.ds(16 * c, 16)] = z16

        arow = sid * RPS
        for j in range(6):
            pltpu.sync_copy(outw, acc_sh.at[pl.ds(arow + j * EB, EB)])
        pltpu.sync_copy(outw.at[pl.ds(0, 25)],
                        acc_sh.at[pl.ds(arow + 6 * EB, 25)])

        @pl.loop(0, CIT)
        def _(i):
            c = i * NS + sid

            @pl.when(c < NCH)
            def _():
                r0 = c * ZR
                pltpu.sync_copy(t_hbm.at[pl.ds(r0, ZR)],
                                t_sh.at[pl.ds(r0, ZR)])

        plsc.subcore_barrier()

        iot = lax.iota(jnp.int32, 16)
        pks = [jnp.int32(64) + lax.shift_right_logical(iot + (16 * k), shift)
               for k in range(4)]

        pltpu.sync_copy(src_hbm.at[wid], sidx)
        pltpu.sync_copy(dst_hbm.at[wid], didx)

        @pl.loop(0, NBLK)
        def _(k):
            pltpu.sync_copy(t_sh.at[sidx.at[k]], outw)
            pltpu.sync_copy(ad_hbm.at[didx.at[k, 0]], adrows)

            @pl.loop(0, EB)
            def _(b):
                s = outw[b, pl.ds(64, 16)] + adrows[b, pl.ds(0, 16)]
                w = _leaky_exp(s)
                outw[b, pl.ds(64, 16)] = w
                rowi = jnp.full((16,), b, jnp.int32)
                for kk in range(4):
                    wb = plsc.load_gather(outw, [rowi, pks[kk]])
                    outw[b, pl.ds(16 * kk, 16)] = (
                        outw[b, pl.ds(16 * kk, 16)] * wb)

            pltpu.sync_copy(outw, acc_sh.at[didx.at[k, 0]], add=True)

        plsc.subcore_barrier()

        @pl.loop(0, CIT)
        def _(i):
            c = i * NS + sid

            @pl.when(c < NCH)
            def _():
                r0 = c * ZR
                pltpu.sync_copy(acc_sh.at[pl.ds(r0, ZR)],
                                acc_hbm.at[cid, pl.ds(r0, ZR)])

    kern = pl.kernel(
        body,
        out_type=jax.ShapeDtypeStruct((NC, N, TW), _f32),
        mesh=mesh,
        scratch_types=[
            pltpu.VMEM_SHARED((N, TW), _f32),
            pltpu.VMEM_SHARED((N, TW), _f32),
            pltpu.VMEM((NBLK, EB), jnp.int32),
            pltpu.VMEM((NBLK, 1, EB), jnp.int32),
            pltpu.VMEM((EB, TW), _f32),
            pltpu.VMEM((EB, 16), _f32),
        ],
        compiler_params=_sc_compiler_params(),
    )
    return kern(t_arr, ad_arr, src.reshape(NW, NBLK, EB),
                dst.reshape(NW, NBLK, 1, EB))


def kernel(x, edge_index, W1, a_src1, a_dst1, b1, W2, a_src2, a_dst2, b2):
    src = edge_index[0]
    dst = edge_index[1]

    eye8 = jnp.eye(8, dtype=_f32)
    A_s1 = (a_src1.reshape(8, 8)[:, :, None] * eye8[:, None, :]).reshape(64, 8)
    A_d1 = (a_dst1.reshape(8, 8)[:, :, None] * eye8[:, None, :]).reshape(64, 8)

    t1, ad1 = _tc_stage_a(x, W1, A_s1, A_d1)
    acc1 = _sc_edge_pass(t1, ad1, src, dst, shift=3)
    t2, ad2 = _tc_stage_b(acc1, t1, ad1, b1.reshape(1, 64), W2,
                          a_src2.reshape(1, 64), a_dst2.reshape(1, 64))
    acc2 = _sc_edge_pass(t2, ad2, src, dst, shift=6)
    return _tc_stage_c(acc2, t2, ad2, b2.reshape(1, 64))


# speedup vs baseline: 47.6172x; 1.1116x over previous
import dataclasses
import functools

import jax
import jax.numpy as jnp
from jax import lax
from jax.experimental import pallas as pl
from jax.experimental.pallas import tpu as pltpu
from jax.experimental.pallas import tpu_sc as plsc

N = 10000
E = 320000
NC = 2
NS = 16
NW = NC * NS
EPW = E // NW
EB = 100
NBLK = EPW // EB
TW = 80
ZR = 16
NCH = N // ZR
CIT = -(-NCH // NS)
RPS = N // NS

_HI = jax.lax.Precision.HIGHEST
_f32 = jnp.float32


def _leaky_exp(s):
    return jnp.exp(jnp.maximum(s, s * jnp.float32(0.2)))


_R = 2000
_G = N // _R


def _tc_stage_a(x, W1, A_s, A_d):
    def body(x_ref, w_ref, as_ref, ad_ref, t_ref, adt_ref):
        h = lax.dot(x_ref[...], w_ref[...], precision=_HI,
                    preferred_element_type=_f32)
        als = lax.dot(h, as_ref[...], precision=_HI,
                      preferred_element_type=_f32)
        ald = lax.dot(h, ad_ref[...], precision=_HI,
                      preferred_element_type=_f32)
        z8 = jnp.zeros((_R, 8), _f32)
        t_ref[...] = jnp.concatenate([h, als, z8], axis=1)
        adt_ref[...] = jnp.concatenate([ald, z8], axis=1)

    return pl.pallas_call(
        body,
        grid=(_G,),
        in_specs=[
            pl.BlockSpec((_R, 128), lambda i: (i, 0)),
            pl.BlockSpec((128, 64), lambda i: (0, 0)),
            pl.BlockSpec((64, 8), lambda i: (0, 0)),
            pl.BlockSpec((64, 8), lambda i: (0, 0)),
        ],
        out_specs=[
            pl.BlockSpec((_R, TW), lambda i: (i, 0)),
            pl.BlockSpec((_R, 16), lambda i: (i, 0)),
        ],
        out_shape=[
            jax.ShapeDtypeStruct((N, TW), _f32),
            jax.ShapeDtypeStruct((N, 16), _f32),
        ],
    )(x, W1, A_s, A_d)


def _tc_stage_b(acc, t1, ad1, b1, W2, a2s, a2d):
    def body(a_ref, t_ref, adt_ref, b_ref, w_ref, a2s_ref, a2d_ref,
             t2_ref, adt2_ref):
        nb = a_ref[0, :, :64] + a_ref[1, :, :64]
        db = a_ref[0, :, 64:72] + a_ref[1, :, 64:72]
        h1 = t_ref[:, :64]
        as1 = t_ref[:, 64:72]
        ad1v = adt_ref[:, :8]
        wself = _leaky_exp(as1 + ad1v)
        parts = []
        for hh in range(8):
            ws = wself[:, hh:hh + 1]
            numh = nb[:, 8 * hh:8 * hh + 8] + h1[:, 8 * hh:8 * hh + 8] * ws
            denh = db[:, hh:hh + 1] + ws
            parts.append(numh / denh)
        out1 = jnp.concatenate(parts, axis=1) + b_ref[...]
        x2 = jnp.where(out1 > 0, out1, jnp.exp(out1) - jnp.float32(1.0))
        h2 = lax.dot(x2, w_ref[...], precision=_HI,
                     preferred_element_type=_f32)
        as2 = jnp.sum(h2 * a2s_ref[...], axis=1, keepdims=True)
        ad2 = jnp.sum(h2 * a2d_ref[...], axis=1, keepdims=True)
        z15 = jnp.zeros((_R, 15), _f32)
        t2_ref[...] = jnp.concatenate([h2, as2, z15], axis=1)
        adt2_ref[...] = jnp.concatenate([ad2, z15], axis=1)

    return pl.pallas_call(
        body,
        grid=(_G,),
        in_specs=[
            pl.BlockSpec((NC, _R, TW), lambda i: (0, i, 0)),
            pl.BlockSpec((_R, TW), lambda i: (i, 0)),
            pl.BlockSpec((_R, 16), lambda i: (i, 0)),
            pl.BlockSpec((1, 64), lambda i: (0, 0)),
            pl.BlockSpec((64, 64), lambda i: (0, 0)),
            pl.BlockSpec((1, 64), lambda i: (0, 0)),
            pl.BlockSpec((1, 64), lambda i: (0, 0)),
        ],
        out_specs=[
            pl.BlockSpec((_R, TW), lambda i: (i, 0)),
            pl.BlockSpec((_R, 16), lambda i: (i, 0)),
        ],
        out_shape=[
            jax.ShapeDtypeStruct((N, TW), _f32),
            jax.ShapeDtypeStruct((N, 16), _f32),
        ],
    )(acc, t1, ad1, b1, W2, a2s, a2d)


def _tc_stage_c(acc, t2, ad2, b2):
    def body(a_ref, t_ref, adt_ref, b_ref, o_ref):
        nb = a_ref[0, :, :64] + a_ref[1, :, :64]
        db = a_ref[0, :, 64:65] + a_ref[1, :, 64:65]
        h2 = t_ref[:, :64]
        as2 = t_ref[:, 64:65]
        ad2v = adt_ref[:, :1]
        wself = _leaky_exp(as2 + ad2v)
        o_ref[...] = (nb + h2 * wself) / (db + wself) + b_ref[...]

    return pl.pallas_call(
        body,
        grid=(_G,),
        in_specs=[
            pl.BlockSpec((NC, _R, TW), lambda i: (0, i, 0)),
            pl.BlockSpec((_R, TW), lambda i: (i, 0)),
            pl.BlockSpec((_R, 16), lambda i: (i, 0)),
            pl.BlockSpec((1, 64), lambda i: (0, 0)),
        ],
        out_specs=pl.BlockSpec((_R, 64), lambda i: (i, 0)),
        out_shape=jax.ShapeDtypeStruct((N, 64), _f32),
    )(acc, t2, ad2, b2)


def _sc_compiler_params():
    cp = pltpu.CompilerParams()
    fields = pltpu.CompilerParams.__dataclass_fields__
    if "needs_layout_passes" in fields:
        cp = dataclasses.replace(cp, needs_layout_passes=False)
    if "use_tc_tiling_on_sc" in fields:
        cp = dataclasses.replace(cp, use_tc_tiling_on_sc=False)
    return cp


@functools.partial(jax.jit, static_argnames=("shift",))
def _sc_edge_pass(t_arr, ad_arr, src, dst, *, shift):
    mesh = plsc.VectorSubcoreMesh(core_axis_name="c", subcore_axis_name="s",
                                  num_cores=NC, num_subcores=NS)

    def body(t_hbm, ad_hbm, src_hbm, dst_hbm, acc_hbm,
             t_sh, acc_sh, sidx, didx, trows, outw, adrows):
        cid = lax.axis_index("c")
        sid = lax.axis_index("s")
        wid = sid * NC + cid

        z16 = jnp.zeros((16,), _f32)

        @pl.loop(0, EB)
        def _(r):
            for c in range(5):
                outw[r, pl.ds(16 * c, 16)] = z16

        arow = sid * RPS
        for j in range(6):
            pltpu.sync_copy(outw, acc_sh.at[pl.ds(arow + j * EB, EB)])
        pltpu.sync_copy(outw.at[pl.ds(0, 25)],
                        acc_sh.at[pl.ds(arow + 6 * EB, 25)])

        @pl.loop(0, CIT)
        def _(i):
            c = i * NS + sid

            @pl.when(c < NCH)
            def _():
                r0 = c * ZR
                pltpu.sync_copy(t_hbm.at[pl.ds(r0, ZR)],
                                t_sh.at[pl.ds(r0, ZR)])

        plsc.subcore_barrier()

        iot = lax.iota(jnp.int32, 16)
        pks = [jnp.int32(64) + lax.shift_right_logical(iot + (16 * k), shift)
               for k in range(4)]

        pltpu.sync_copy(src_hbm.at[wid], sidx)
        pltpu.sync_copy(dst_hbm.at[wid], didx)

        @pl.loop(0, NBLK)
        def _(k):
            pltpu.sync_copy(t_sh.at[sidx.at[k]], trows)
            pltpu.sync_copy(ad_hbm.at[didx.at[k, 0]], adrows)

            @pl.loop(0, EB)
            def _(b):
                a_s, _ = plsc.unpack(trows[b, pl.ds(64, 32)],
                                     format=plsc.PackFormat.INTERLEAVED,
                                     preferred_element_type=_f32)
                s = a_s + adrows[b, pl.ds(0, 16)]
                w = _leaky_exp(s)
                outw[b, pl.ds(64, 16)] = w
                rowi = jnp.full((16,), b, jnp.int32)
                for kk in range(2):
                    ha, hb = plsc.unpack(trows[b, pl.ds(32 * kk, 32)],
                                         format=plsc.PackFormat.INTERLEAVED,
                                         preferred_element_type=_f32)
                    wba = plsc.load_gather(outw, [rowi, pks[2 * kk]])
                    wbb = plsc.load_gather(outw, [rowi, pks[2 * kk + 1]])
                    outw[b, pl.ds(32 * kk, 16)] = ha * wba
                    outw[b, pl.ds(32 * kk + 16, 16)] = hb * wbb

            pltpu.sync_copy(outw, acc_sh.at[didx.at[k, 0]], add=True)

        plsc.subcore_barrier()

        @pl.loop(0, CIT)
        def _(i):
            c = i * NS + sid

            @pl.when(c < NCH)
            def _():
                r0 = c * ZR
                pltpu.sync_copy(acc_sh.at[pl.ds(r0, ZR)],
                                acc_hbm.at[cid, pl.ds(r0, ZR)])

    kern = pl.kernel(
        body,
        out_type=jax.ShapeDtypeStruct((NC, N, TW), _f32),
        mesh=mesh,
        scratch_types=[
            pltpu.VMEM_SHARED((N, 96), jnp.bfloat16),
            pltpu.VMEM_SHARED((N, TW), _f32),
            pltpu.VMEM((NBLK, EB), jnp.int32),
            pltpu.VMEM((NBLK, 1, EB), jnp.int32),
            pltpu.VMEM((EB, 96), jnp.bfloat16),
            pltpu.VMEM((EB, TW), _f32),
            pltpu.VMEM((EB, 16), _f32),
        ],
        compiler_params=_sc_compiler_params(),
    )
    return kern(t_arr, ad_arr, src.reshape(NW, NBLK, EB),
                dst.reshape(NW, NBLK, 1, EB))


def _to_bf16_table(t):
    x96 = jnp.concatenate([t, jnp.zeros((N, 16), _f32)], axis=1)
    x96 = x96.reshape(N, 3, 2, 16).swapaxes(2, 3).reshape(N, 96)
    return x96.astype(jnp.bfloat16)


def kernel(x, edge_index, W1, a_src1, a_dst1, b1, W2, a_src2, a_dst2, b2):
    src = edge_index[0]
    dst = edge_index[1]

    eye8 = jnp.eye(8, dtype=_f32)
    A_s1 = (a_src1.reshape(8, 8)[:, :, None] * eye8[:, None, :]).reshape(64, 8)
    A_d1 = (a_dst1.reshape(8, 8)[:, :, None] * eye8[:, None, :]).reshape(64, 8)

    t1, ad1 = _tc_stage_a(x, W1, A_s1, A_d1)
    acc1 = _sc_edge_pass(_to_bf16_table(t1), ad1, src, dst, shift=3)
    t2, ad2 = _tc_stage_b(acc1, t1, ad1, b1.reshape(1, 64), W2,
                          a_src2.reshape(1, 64), a_dst2.reshape(1, 64))
    acc2 = _sc_edge_pass(_to_bf16_table(t2), ad2, src, dst, shift=6)
    return _tc_stage_c(acc2, t2, ad2, b2.reshape(1, 64))
